# baseline (device time: 27499 ns/iter reference)
import jax
import jax.numpy as jnp
from jax import lax
from jax.experimental import pallas as pl
from jax.experimental.pallas import tpu as pltpu

N_DEV = 32


def kernel(t, W):
    m, k = t.shape
    n = W.shape[1]
    ch = m // N_DEV

    def body(t_ref, w_ref, out_ref, comm_ref,
             rs_ssem, rs_rsem, ag_ssem, ag_rsem):
        my = lax.axis_index("i")

        comm_ref[0, :, :] = t_ref[pl.ds(my * ch, ch), :]

        barrier = pltpu.get_barrier_semaphore()
        for kk in range(1, N_DEV):
            peer = jnp.mod(my + kk, N_DEV)
            pl.semaphore_signal(
                barrier, inc=1,
                device_id=(peer,), device_id_type=pl.DeviceIdType.MESH,
            )
        pl.semaphore_wait(barrier, N_DEV - 1)

        rs_sends = []
        for kk in range(1, N_DEV):
            peer = jnp.mod(my + kk, N_DEV)
            rdma = pltpu.make_async_remote_copy(
                src_ref=t_ref.at[pl.ds(peer * ch, ch)],
                dst_ref=comm_ref.at[N_DEV - kk],
                send_sem=rs_ssem.at[kk],
                recv_sem=rs_rsem.at[N_DEV - kk],
                device_id=(peer,),
                device_id_type=pl.DeviceIdType.MESH,
            )
            rdma.start()
            rs_sends.append(rdma)

        for j in range(1, N_DEV):
            recv = pltpu.make_async_remote_copy(
                src_ref=comm_ref.at[j],
                dst_ref=comm_ref.at[j],
                send_sem=rs_ssem.at[j],
                recv_sem=rs_rsem.at[j],
                device_id=(my,),
                device_id_type=pl.DeviceIdType.MESH,
            )
            recv.wait_recv()
        acc = jnp.sum(comm_ref[:, :, :], axis=0)

        y = jnp.dot(acc, w_ref[:, :], preferred_element_type=jnp.float32)
        out_ref[pl.ds(my * ch, ch), :] = y

        ag_sends = []
        for kk in range(1, N_DEV):
            peer = jnp.mod(my + kk, N_DEV)
            rdma = pltpu.make_async_remote_copy(
                src_ref=out_ref.at[pl.ds(my * ch, ch)],
                dst_ref=out_ref.at[pl.ds(my * ch, ch)],
                send_sem=ag_ssem.at[kk],
                recv_sem=ag_rsem.at[N_DEV - kk],
                device_id=(peer,),
                device_id_type=pl.DeviceIdType.MESH,
            )
            rdma.start()
            ag_sends.append(rdma)

        for rdma in rs_sends:
            rdma.wait_send()

        for j in range(1, N_DEV):
            src_dev = jnp.mod(my + j, N_DEV)
            recv = pltpu.make_async_remote_copy(
                src_ref=out_ref.at[pl.ds(src_dev * ch, ch)],
                dst_ref=out_ref.at[pl.ds(src_dev * ch, ch)],
                send_sem=ag_ssem.at[j],
                recv_sem=ag_rsem.at[j],
                device_id=(my,),
                device_id_type=pl.DeviceIdType.MESH,
            )
            recv.wait_recv()

        for rdma in ag_sends:
            rdma.wait_send()

    return pl.pallas_call(
        body,
        out_shape=jax.ShapeDtypeStruct((m, n), jnp.float32),
        in_specs=[
            pl.BlockSpec(memory_space=pltpu.VMEM),
            pl.BlockSpec(memory_space=pltpu.VMEM),
        ],
        out_specs=pl.BlockSpec(memory_space=pltpu.VMEM),
        scratch_shapes=[
            pltpu.VMEM((N_DEV, ch, k), jnp.float32),
            pltpu.SemaphoreType.DMA((N_DEV,)),
            pltpu.SemaphoreType.DMA((N_DEV,)),
            pltpu.SemaphoreType.DMA((N_DEV,)),
            pltpu.SemaphoreType.DMA((N_DEV,)),
        ],
        compiler_params=pltpu.CompilerParams(collective_id=0),
    )(t, W)


# device time: 23879 ns/iter; 1.1516x vs baseline; 1.1516x over previous
import jax
import jax.numpy as jnp
from jax import lax
from jax.experimental import pallas as pl
from jax.experimental.pallas import tpu as pltpu

N_DEV = 32


def kernel(t, W):
    m, k = t.shape
    n = W.shape[1]
    ch = m // N_DEV

    def body(t_ref, w_ref, out_ref, tb_ref, comm_ref, ag_ref,
             rs_ssem, rs_rsem, ag_ssem, ag_rsem):
        my = lax.axis_index("i")

        tb_ref[:, :] = t_ref[:, :].astype(jnp.bfloat16)

        barrier = pltpu.get_barrier_semaphore()
        for kk in range(1, N_DEV):
            peer = jnp.mod(my + kk, N_DEV)
            pl.semaphore_signal(
                barrier, inc=1,
                device_id=(peer,), device_id_type=pl.DeviceIdType.MESH,
            )
        pl.semaphore_wait(barrier, N_DEV - 1)

        rs_sends = []
        for kk in range(1, N_DEV):
            peer = jnp.mod(my + kk, N_DEV)
            rdma = pltpu.make_async_remote_copy(
                src_ref=tb_ref.at[pl.ds(peer * ch, ch)],
                dst_ref=comm_ref.at[N_DEV - kk],
                send_sem=rs_ssem.at[kk],
                recv_sem=rs_rsem.at[N_DEV - kk],
                device_id=(peer,),
                device_id_type=pl.DeviceIdType.MESH,
            )
            rdma.start()
            rs_sends.append(rdma)

        for j in range(1, N_DEV):
            recv = pltpu.make_async_remote_copy(
                src_ref=comm_ref.at[j],
                dst_ref=comm_ref.at[j],
                send_sem=rs_ssem.at[j],
                recv_sem=rs_rsem.at[j],
                device_id=(my,),
                device_id_type=pl.DeviceIdType.MESH,
            )
            recv.wait_recv()
        acc = t_ref[pl.ds(my * ch, ch), :] + jnp.sum(
            comm_ref[pl.ds(1, N_DEV - 1), :, :].astype(jnp.float32), axis=0
        )

        y = jnp.dot(acc, w_ref[:, :], preferred_element_type=jnp.float32)
        ag_ref[pl.ds(my, 1), :, :] = y.astype(jnp.bfloat16)[None]

        ag_sends = []
        for kk in range(1, N_DEV):
            peer = jnp.mod(my + kk, N_DEV)
            rdma = pltpu.make_async_remote_copy(
                src_ref=ag_ref.at[pl.ds(my, 1)],
                dst_ref=ag_ref.at[pl.ds(my, 1)],
                send_sem=ag_ssem.at[kk],
                recv_sem=ag_rsem.at[N_DEV - kk],
                device_id=(peer,),
                device_id_type=pl.DeviceIdType.MESH,
            )
            rdma.start()
            ag_sends.append(rdma)

        for rdma in rs_sends:
            rdma.wait_send()

        for j in range(1, N_DEV):
            src_dev = jnp.mod(my + j, N_DEV)
            recv = pltpu.make_async_remote_copy(
                src_ref=ag_ref.at[pl.ds(src_dev, 1)],
                dst_ref=ag_ref.at[pl.ds(src_dev, 1)],
                send_sem=ag_ssem.at[j],
                recv_sem=ag_rsem.at[j],
                device_id=(my,),
                device_id_type=pl.DeviceIdType.MESH,
            )
            recv.wait_recv()

        out_ref[:, :] = ag_ref[:, :, :].astype(jnp.float32).reshape(m, n)

        for rdma in ag_sends:
            rdma.wait_send()

    return pl.pallas_call(
        body,
        out_shape=jax.ShapeDtypeStruct((m, n), jnp.float32),
        in_specs=[
            pl.BlockSpec(memory_space=pltpu.VMEM),
            pl.BlockSpec(memory_space=pltpu.VMEM),
        ],
        out_specs=pl.BlockSpec(memory_space=pltpu.VMEM),
        scratch_shapes=[
            pltpu.VMEM((m, k), jnp.bfloat16),
            pltpu.VMEM((N_DEV, ch, k), jnp.bfloat16),
            pltpu.VMEM((N_DEV, ch, n), jnp.bfloat16),
            pltpu.SemaphoreType.DMA((N_DEV,)),
            pltpu.SemaphoreType.DMA((N_DEV,)),
            pltpu.SemaphoreType.DMA((N_DEV,)),
            pltpu.SemaphoreType.DMA((N_DEV,)),
        ],
        compiler_params=pltpu.CompilerParams(collective_id=0),
    )(t, W)


# device time: 21761 ns/iter; 1.2637x vs baseline; 1.0973x over previous
import jax
import jax.numpy as jnp
from jax import lax
from jax.experimental import pallas as pl
from jax.experimental.pallas import tpu as pltpu

N_DEV = 32

_SNAKE = [(0, 0), (1, 0), (1, 1), (0, 1), (0, 2), (1, 2), (1, 3), (0, 3)]
_COORDS = [(x, y, z) for z in range(4) for (x, y) in _SNAKE]
_ID_OF = {c: i for i, c in enumerate(_COORDS)}
_Y = [c[1] for c in _COORDS]
_R = [c[0] * 4 + c[2] for c in _COORDS]
_G = [y * 8 + r for y, r in zip(_Y, _R)]
_P1 = [
    [_ID_OF[(x, (y + kk) % 4, z)] for kk in (1, 2, 3)]
    for (x, y, z) in _COORDS
]
_P2 = [
    [
        _ID_OF[(((_R[i] + kk) % 8) // 4, _COORDS[i][1], ((_R[i] + kk) % 8) % 4)]
        for kk in range(1, 8)
    ]
    for i in range(N_DEV)
]


def kernel(t, W):
    m, k = t.shape
    n = W.shape[1]
    ch = m // N_DEV
    pc = m // 4

    def body(t_ref, w_ref, out_ref, tb_ref, s1_ref, py_ref, s2_ref, ag_ref,
             s1_ssem, s1_rsem, s2_ssem, s2_rsem,
             agA_ssem, agA_rsem, agB_ssem, agB_rsem):
        my = lax.axis_index("i")
        z = my // 8
        o = jnp.mod(my, 8)
        y = o // 2
        x = jnp.mod(jnp.mod(o, 2) + jnp.mod(y, 2), 2)
        r = x * 4 + z
        g = y * 8 + r

        def id_of(xx, yy, zz):
            return zz * 8 + 2 * yy + jnp.mod(xx + yy, 2)

        p1 = [id_of(x, jnp.mod(y + kk, 4), z) for kk in (1, 2, 3)]
        p2 = []
        for kk in range(1, 8):
            rp = jnp.mod(r + kk, 8)
            p2.append(id_of(rp // 4, y, jnp.mod(rp, 4)))

        tb_ref[:, :] = t_ref[:, :].astype(jnp.bfloat16)

        barrier = pltpu.get_barrier_semaphore()
        for kk in range(3):
            pl.semaphore_signal(
                barrier, inc=1,
                device_id=(p1[kk],), device_id_type=pl.DeviceIdType.MESH,
            )
        for kk in range(7):
            pl.semaphore_signal(
                barrier, inc=1,
                device_id=(p2[kk],), device_id_type=pl.DeviceIdType.MESH,
            )
        pl.semaphore_wait(barrier, 10)

        rs_sends = []
        for kk in range(1, 4):
            py_peer = jnp.mod(y + kk, 4)
            rdma = pltpu.make_async_remote_copy(
                src_ref=tb_ref.at[pl.ds(py_peer * pc, pc)],
                dst_ref=s1_ref.at[4 - kk],
                send_sem=s1_ssem.at[kk],
                recv_sem=s1_rsem.at[4 - kk],
                device_id=(p1[kk - 1],),
                device_id_type=pl.DeviceIdType.MESH,
            )
            rdma.start()
            rs_sends.append(rdma)
        for j in range(1, 4):
            pltpu.make_async_remote_copy(
                src_ref=s1_ref.at[j], dst_ref=s1_ref.at[j],
                send_sem=s1_ssem.at[j], recv_sem=s1_rsem.at[j],
                device_id=(my,), device_id_type=pl.DeviceIdType.MESH,
            ).wait_recv()
        tot = t_ref[pl.ds(y * pc, pc), :] + jnp.sum(
            s1_ref[pl.ds(1, 3), :, :].astype(jnp.float32), axis=0
        )
        py_ref[:, :] = tot.astype(jnp.bfloat16)

        for kk in range(1, 8):
            pr = jnp.mod(r + kk, 8)
            rdma = pltpu.make_async_remote_copy(
                src_ref=py_ref.at[pl.ds(pr * ch, ch)],
                dst_ref=s2_ref.at[8 - kk],
                send_sem=s2_ssem.at[kk],
                recv_sem=s2_rsem.at[8 - kk],
                device_id=(p2[kk - 1],),
                device_id_type=pl.DeviceIdType.MESH,
            )
            rdma.start()
            rs_sends.append(rdma)
        for j in range(1, 8):
            pltpu.make_async_remote_copy(
                src_ref=s2_ref.at[j], dst_ref=s2_ref.at[j],
                send_sem=s2_ssem.at[j], recv_sem=s2_rsem.at[j],
                device_id=(my,), device_id_type=pl.DeviceIdType.MESH,
            ).wait_recv()
        acc = py_ref[pl.ds(r * ch, ch), :].astype(jnp.float32) + jnp.sum(
            s2_ref[pl.ds(1, 7), :, :].astype(jnp.float32), axis=0
        )

        ymat = jnp.dot(acc, w_ref[:, :], preferred_element_type=jnp.float32)
        ag_ref[pl.ds(g, 1), :, :] = ymat.astype(jnp.bfloat16)[None]

        ag_sends = []
        for kk in range(1, 8):
            rdma = pltpu.make_async_remote_copy(
                src_ref=ag_ref.at[pl.ds(g, 1)],
                dst_ref=ag_ref.at[pl.ds(g, 1)],
                send_sem=agA_ssem.at[kk],
                recv_sem=agA_rsem.at[8 - kk],
                device_id=(p2[kk - 1],),
                device_id_type=pl.DeviceIdType.MESH,
            )
            rdma.start()
            ag_sends.append(rdma)

        for rdma in rs_sends:
            rdma.wait_send()

        for j in range(1, 8):
            gs = y * 8 + jnp.mod(r + j, 8)
            pltpu.make_async_remote_copy(
                src_ref=ag_ref.at[pl.ds(gs, 1)], dst_ref=ag_ref.at[pl.ds(gs, 1)],
                send_sem=agA_ssem.at[j], recv_sem=agA_rsem.at[j],
                device_id=(my,), device_id_type=pl.DeviceIdType.MESH,
            ).wait_recv()

        for kk in range(1, 4):
            rdma = pltpu.make_async_remote_copy(
                src_ref=ag_ref.at[pl.ds(y * 8, 8)],
                dst_ref=ag_ref.at[pl.ds(y * 8, 8)],
                send_sem=agB_ssem.at[kk],
                recv_sem=agB_rsem.at[4 - kk],
                device_id=(p1[kk - 1],),
                device_id_type=pl.DeviceIdType.MESH,
            )
            rdma.start()
            ag_sends.append(rdma)
        for j in range(1, 4):
            yb = jnp.mod(y + j, 4)
            pltpu.make_async_remote_copy(
                src_ref=ag_ref.at[pl.ds(yb * 8, 8)],
                dst_ref=ag_ref.at[pl.ds(yb * 8, 8)],
                send_sem=agB_ssem.at[j], recv_sem=agB_rsem.at[j],
                device_id=(my,), device_id_type=pl.DeviceIdType.MESH,
            ).wait_recv()

        out_ref[:, :] = ag_ref[:, :, :].astype(jnp.float32).reshape(m, n)

        for rdma in ag_sends:
            rdma.wait_send()

    return pl.pallas_call(
        body,
        out_shape=jax.ShapeDtypeStruct((m, n), jnp.float32),
        in_specs=[
            pl.BlockSpec(memory_space=pltpu.VMEM),
            pl.BlockSpec(memory_space=pltpu.VMEM),
        ],
        out_specs=pl.BlockSpec(memory_space=pltpu.VMEM),
        scratch_shapes=[
            pltpu.VMEM((m, k), jnp.bfloat16),
            pltpu.VMEM((4, pc, k), jnp.bfloat16),
            pltpu.VMEM((pc, k), jnp.bfloat16),
            pltpu.VMEM((8, ch, k), jnp.bfloat16),
            pltpu.VMEM((N_DEV, ch, n), jnp.bfloat16),
            pltpu.SemaphoreType.DMA((4,)),
            pltpu.SemaphoreType.DMA((4,)),
            pltpu.SemaphoreType.DMA((8,)),
            pltpu.SemaphoreType.DMA((8,)),
            pltpu.SemaphoreType.DMA((8,)),
            pltpu.SemaphoreType.DMA((8,)),
            pltpu.SemaphoreType.DMA((4,)),
            pltpu.SemaphoreType.DMA((4,)),
        ],
        compiler_params=pltpu.CompilerParams(collective_id=0),
    )(t, W)
